# baseline (speedup 1.0000x reference)
import functools

import jax
import jax.numpy as jnp
from jax import lax
from jax.experimental import pallas as pl
from jax.experimental.pallas import tpu as pltpu
from jax.experimental.pallas import tpu_sc as plsc

NC = 2
NS = 16
NW = NC * NS
CHUNK = 128
GK = 4
BR = 512
LW = 128


def _mesh():
  return plsc.VectorSubcoreMesh(core_axis_name="c", subcore_axis_name="s")


@functools.lru_cache(maxsize=None)
def _deg_kernel(NP, J):
  rpt = NP // NS

  def body(dstp_hbm, out_hbm, dst_idx, ones_b, lin, deg_sh):
    c = lax.axis_index("c")
    s = lax.axis_index("s")
    w = c * NS + s
    base = s * rpt
    pltpu.sync_copy(dstp_hbm.at[w], dst_idx)
    for k in range(CHUNK // 16):
      ones_b[pl.ds(k * 16, 16)] = jnp.ones((16,), jnp.float32)
    for k in range(rpt // 16):
      lin[pl.ds(k * 16, 16)] = jnp.zeros((16,), jnp.float32)
    pltpu.sync_copy(lin, deg_sh.at[pl.ds(base, rpt)])
    plsc.subcore_barrier()

    def step(j, carry):
      pltpu.sync_copy(ones_b, deg_sh.at[dst_idx.at[j]], add=True)
      return carry

    lax.fori_loop(0, J, step, 0)
    plsc.subcore_barrier()
    pltpu.sync_copy(deg_sh.at[pl.ds(base, rpt)], lin)
    pltpu.sync_copy(lin, out_hbm.at[c].at[pl.ds(base, rpt)])

  return pl.kernel(
      body,
      out_type=jax.ShapeDtypeStruct((NC, NP), jnp.float32),
      mesh=_mesh(),
      scratch_types=[
          pltpu.VMEM((J, CHUNK), jnp.int32),
          pltpu.VMEM((CHUNK,), jnp.float32),
          pltpu.VMEM((rpt,), jnp.float32),
          pltpu.VMEM_SHARED((NP,), jnp.float32),
      ],
  )


@functools.lru_cache(maxsize=None)
def _edge_kernel(NP, J, D):
  rpt = NP // NS
  wb = rpt // CHUNK
  NG = J // GK

  def body(z_hbm, ep_hbm, out_hbm,
           ibufA, ibufB, rows0, rows1, acc, semiA, semiB,
           sem0, sem1, ssem0, ssem1):
    c = lax.axis_index("c")
    s = lax.axis_index("s")
    w = c * NS + s
    base = s * rpt
    pltpu.async_copy(ep_hbm.at[w].at[0], ibufA, semiA)

    def zr(r, carry):
      for k in range(D // 16):
        rows0[r, pl.ds(k * 16, 16)] = jnp.zeros((16,), jnp.float32)
      return carry

    lax.fori_loop(0, CHUNK, zr, 0)
    for k in range(wb):
      pltpu.sync_copy(rows0, acc.at[pl.ds(base + k * CHUNK, CHUNK)])
    plsc.subcore_barrier()

    rows = (rows0, rows1)
    gsem = (sem0, sem1)
    ssem = (ssem0, ssem1)

    pltpu.make_async_copy(ep_hbm.at[w].at[0], ibufA, semiA).wait()
    pltpu.async_copy(z_hbm.at[ibufA.at[0].at[0]], rows0, sem0)

    def half(cur, nxt, semi_nxt, nxt_g, guard, refill, first_wait_guard):
      for k in range(GK):
        p = k % 2
        rb, gs = rows[p], gsem[p]
        nrb, ngs = rows[1 - p], gsem[1 - p]
        pltpu.make_async_copy(z_hbm.at[cur.at[k].at[0]], rb, gs).wait()
        pltpu.make_async_copy(rb, acc.at[cur.at[k].at[1]], ssem[p]).start(add=True)
        def _retire_prev():
          pltpu.make_async_copy(nrb, acc.at[cur.at[k].at[1]], ssem[1 - p]).wait()
        if k == 0 and first_wait_guard is not None:
          pl.when(first_wait_guard)(_retire_prev)
        else:
          _retire_prev()
        if k == 0:
          tbuf, tsem, tg, tguard = refill
          def _refill():
            pltpu.async_copy(ep_hbm.at[w].at[tg], tbuf, tsem)
          if tguard is None:
            _refill()
          else:
            pl.when(tguard)(_refill)
        if k < GK - 1:
          pltpu.async_copy(z_hbm.at[cur.at[k + 1].at[0]], nrb, ngs)
        else:
          def _start_next():
            pltpu.make_async_copy(ep_hbm.at[w].at[nxt_g], nxt, semi_nxt).wait()
            pltpu.async_copy(z_hbm.at[nxt.at[0].at[0]], nrb, ngs)
          if guard is None:
            _start_next()
          else:
            pl.when(guard)(_start_next)

    def step(i, carry):
      more = i < NG // 2 - 1
      half(ibufA, ibufB, semiB, 2 * i + 1, None,
           (ibufB, semiB, 2 * i + 1, None), i > 0)
      half(ibufB, ibufA, semiA, 2 * i + 2, more,
           (ibufA, semiA, 2 * i + 2, more), None)
      return carry

    lax.fori_loop(0, NG // 2, step, 0)
    pltpu.make_async_copy(rows1, acc.at[ibufB.at[GK - 1].at[1]], ssem1).wait()
    plsc.subcore_barrier()
    for k in range(wb):
      sl = pl.ds(base + k * CHUNK, CHUNK)
      pltpu.sync_copy(acc.at[sl], rows0)
      pltpu.sync_copy(rows0, out_hbm.at[c].at[sl])

  return pl.kernel(
      body,
      out_type=jax.ShapeDtypeStruct((NC, NP, D), jnp.float32),
      mesh=_mesh(),
      scratch_types=[
          pltpu.VMEM((GK, 2, CHUNK), jnp.int32),
          pltpu.VMEM((GK, 2, CHUNK), jnp.int32),
          pltpu.VMEM((CHUNK, D), jnp.float32),
          pltpu.VMEM((CHUNK, D), jnp.float32),
          pltpu.VMEM_SHARED((NP, D), jnp.float32),
          pltpu.SemaphoreType.DMA,
          pltpu.SemaphoreType.DMA,
          pltpu.SemaphoreType.DMA,
          pltpu.SemaphoreType.DMA,
          pltpu.SemaphoreType.DMA,
          pltpu.SemaphoreType.DMA,
      ],
  )


def _dot(a, b):
  return jnp.dot(a, b, preferred_element_type=jnp.float32,
                 precision=lax.Precision.HIGHEST)


def _tc_mm(x_pad, W1):
  NP, D = x_pad.shape
  H = W1.shape[1]

  def body(x_ref, w_ref, o_ref):
    o_ref[...] = _dot(x_ref[...], w_ref[...])

  return pl.pallas_call(
      body,
      grid=(NP // BR,),
      in_specs=[
          pl.BlockSpec((BR, D), lambda i: (i, 0)),
          pl.BlockSpec((D, H), lambda i: (0, 0)),
      ],
      out_specs=pl.BlockSpec((BR, H), lambda i: (i, 0)),
      out_shape=jax.ShapeDtypeStruct((NP, H), jnp.float32),
  )(x_pad, W1)


def _tc1(xw1, deg_parts):
  NP, H = xw1.shape

  def body(xw_ref, dp_ref, z_ref, dinv_ref):
    deg = jnp.sum(dp_ref[...], axis=0) + 1.0
    dinv = lax.rsqrt(deg)
    z_ref[...] = xw_ref[...] * dinv[:, None]
    dinv_ref[...] = dinv

  return pl.pallas_call(
      body,
      grid=(NP // BR,),
      in_specs=[
          pl.BlockSpec((BR, H), lambda i: (i, 0)),
          pl.BlockSpec((NC, BR), lambda i: (0, i)),
      ],
      out_specs=[
          pl.BlockSpec((BR, H), lambda i: (i, 0)),
          pl.BlockSpec((BR,), lambda i: (i,)),
      ],
      out_shape=[
          jax.ShapeDtypeStruct((NP, H), jnp.float32),
          jax.ShapeDtypeStruct((NP,), jnp.float32),
      ],
  )(xw1, deg_parts)


def _tc2(parts1, z1, dinv, b1, W2, n_valid):
  _, NP, D = parts1.shape
  H = W2.shape[1]

  def body(p_ref, z1_ref, dinv_ref, b1_ref, w2_ref, h1_ref, z2_ref):
    i = pl.program_id(0)
    sacc = p_ref[0] + p_ref[1] + z1_ref[...]
    dinv_c = dinv_ref[...][:, None]
    h1 = jnp.maximum(sacc * dinv_c + b1_ref[...][None, :], 0.0)
    h1_ref[...] = h1
    row = i * BR + lax.broadcasted_iota(jnp.int32, (BR, 1), 0)
    z2 = _dot(h1, w2_ref[...]) * dinv_c
    z2_ref[...] = jnp.where(row < n_valid, z2, 0.0)

  return pl.pallas_call(
      body,
      grid=(NP // BR,),
      in_specs=[
          pl.BlockSpec((NC, BR, D), lambda i: (0, i, 0)),
          pl.BlockSpec((BR, D), lambda i: (i, 0)),
          pl.BlockSpec((BR,), lambda i: (i,)),
          pl.BlockSpec((D,), lambda i: (0,)),
          pl.BlockSpec((D, H), lambda i: (0, 0)),
      ],
      out_specs=[
          pl.BlockSpec((BR, H), lambda i: (i, 0)),
          pl.BlockSpec((BR, H), lambda i: (i, 0)),
      ],
      out_shape=[
          jax.ShapeDtypeStruct((NP, H), jnp.float32),
          jax.ShapeDtypeStruct((NP, H), jnp.float32),
      ],
  )(parts1, z1, dinv, b1, W2)


def _tc3a(h1, alpha, Wc_p, Wf_p, bc_p, bf_p):
  NP, D = h1.shape

  def body(h1_ref, a_ref, wc_ref, wf_ref, bc_ref, bf_ref, out_ref):
    h1 = h1_ref[...]
    a = a_ref[...][:, None]
    lc = _dot(h1, wc_ref[...])
    lf = _dot((1.0 - a) * h1, wf_ref[...])
    out_ref[...] = 0.5 * (lc + lf + bc_ref[...][None, :] + bf_ref[...][None, :])

  return pl.pallas_call(
      body,
      grid=(NP // BR,),
      in_specs=[
          pl.BlockSpec((BR, D), lambda i: (i, 0)),
          pl.BlockSpec((BR,), lambda i: (i,)),
          pl.BlockSpec((D, LW), lambda i: (0, 0)),
          pl.BlockSpec((D, LW), lambda i: (0, 0)),
          pl.BlockSpec((LW,), lambda i: (0,)),
          pl.BlockSpec((LW,), lambda i: (0,)),
      ],
      out_specs=pl.BlockSpec((BR, LW), lambda i: (i, 0)),
      out_shape=jax.ShapeDtypeStruct((NP, LW), jnp.float32),
  )(h1, alpha, Wc_p, Wf_p, bc_p, bf_p)


def _tc3b(parts2, z2, dinv, b2, alpha, Wf_p, base):
  _, NP, D = parts2.shape

  def body(p_ref, z2_ref, dinv_ref, b2_ref, a_ref, wf_ref, base_ref, out_ref):
    dinv_c = dinv_ref[...][:, None]
    h2 = jnp.maximum((p_ref[0] + p_ref[1] + z2_ref[...]) * dinv_c
                     + b2_ref[...][None, :], 0.0)
    a = a_ref[...][:, None]
    out_ref[...] = base_ref[...] + 0.5 * _dot(a * h2, wf_ref[...])

  return pl.pallas_call(
      body,
      grid=(NP // BR,),
      in_specs=[
          pl.BlockSpec((NC, BR, D), lambda i: (0, i, 0)),
          pl.BlockSpec((BR, D), lambda i: (i, 0)),
          pl.BlockSpec((BR,), lambda i: (i,)),
          pl.BlockSpec((D,), lambda i: (0,)),
          pl.BlockSpec((BR,), lambda i: (i,)),
          pl.BlockSpec((D, LW), lambda i: (0, 0)),
          pl.BlockSpec((BR, LW), lambda i: (i, 0)),
      ],
      out_specs=pl.BlockSpec((BR, LW), lambda i: (i, 0)),
      out_shape=jax.ShapeDtypeStruct((NP, LW), jnp.float32),
  )(parts2, z2, dinv, b2, alpha, Wf_p, base)


def kernel(x, edge_index, h_node, W1, b1, Wc, bc, W2, b2, Wf, bf):
  N, D = x.shape
  H = W1.shape[1]
  E = edge_index.shape[1]
  C = Wc.shape[1]

  NP = -(-N // (NS * CHUNK)) * (NS * CHUNK)
  if NP == N:
    NP += NS * CHUNK
  J = -(-E // (NW * CHUNK))
  J = -(-J // (2 * GK)) * (2 * GK)
  EP = NW * J * CHUNK
  npad = EP - E

  src = edge_index[0].astype(jnp.int32)
  dst = edge_index[1].astype(jnp.int32)
  pad_idx = (N + jnp.arange(npad, dtype=jnp.int32) % (NP - N))
  srcp = jnp.concatenate([src, pad_idx]).reshape(NW, J, CHUNK)
  dstp = jnp.concatenate([dst, pad_idx]).reshape(NW, J, CHUNK)
  ep = jnp.stack([srcp, dstp], axis=2).reshape(NW, J // GK, GK, 2, CHUNK)

  x_pad = jnp.pad(x, ((0, NP - N), (0, 0)))
  a_pad = jnp.pad(h_node, (0, NP - N))
  Wc_p = jnp.pad(Wc, ((0, 0), (0, LW - C)))
  Wf_p = jnp.pad(Wf, ((0, 0), (0, LW - C)))
  bc_p = jnp.pad(bc, (0, LW - C))
  bf_p = jnp.pad(bf, (0, LW - C))

  deg_parts = _deg_kernel(NP, J)(dstp)
  xw1 = _tc_mm(x_pad, W1)
  z1, dinv = _tc1(xw1, deg_parts)
  seg = _edge_kernel(NP, J, H)
  parts1 = seg(z1, ep)
  h1, z2 = _tc2(parts1, z1, dinv, b1, W2, N)
  parts2 = seg(z2, ep)
  base = _tc3a(h1, a_pad, Wc_p, Wf_p, bc_p, bf_p)
  logits_pad = _tc3b(parts2, z2, dinv, b2, a_pad, Wf_p, base)
  return logits_pad[:N, :C]


# speedup vs baseline: 1.0214x; 1.0214x over previous
import functools

import jax
import jax.numpy as jnp
from jax import lax
from jax.experimental import pallas as pl
from jax.experimental.pallas import tpu as pltpu
from jax.experimental.pallas import tpu_sc as plsc

NC = 2
NS = 16
NW = NC * NS
CHUNK = 128
GK = 4
BR = 512
LW = 128


def _mesh():
  return plsc.VectorSubcoreMesh(core_axis_name="c", subcore_axis_name="s")


@functools.lru_cache(maxsize=None)
def _deg_kernel(NP, J):
  rpt = NP // NS

  def body(dstp_hbm, out_hbm, dst_idx, ones_b, lin, deg_sh):
    c = lax.axis_index("c")
    s = lax.axis_index("s")
    w = c * NS + s
    base = s * rpt
    pltpu.sync_copy(dstp_hbm.at[w], dst_idx)
    for k in range(CHUNK // 16):
      ones_b[pl.ds(k * 16, 16)] = jnp.ones((16,), jnp.float32)
    for k in range(rpt // 16):
      lin[pl.ds(k * 16, 16)] = jnp.zeros((16,), jnp.float32)
    pltpu.sync_copy(lin, deg_sh.at[pl.ds(base, rpt)])
    plsc.subcore_barrier()

    def step(j, carry):
      pltpu.sync_copy(ones_b, deg_sh.at[dst_idx.at[j]], add=True)
      return carry

    lax.fori_loop(0, J, step, 0)
    plsc.subcore_barrier()
    pltpu.sync_copy(deg_sh.at[pl.ds(base, rpt)], lin)
    pltpu.sync_copy(lin, out_hbm.at[c].at[pl.ds(base, rpt)])

  return pl.kernel(
      body,
      out_type=jax.ShapeDtypeStruct((NC, NP), jnp.float32),
      mesh=_mesh(),
      scratch_types=[
          pltpu.VMEM((J, CHUNK), jnp.int32),
          pltpu.VMEM((CHUNK,), jnp.float32),
          pltpu.VMEM((rpt,), jnp.float32),
          pltpu.VMEM_SHARED((NP,), jnp.float32),
      ],
  )


@functools.lru_cache(maxsize=None)
def _edge_kernel(NP, J, D):
  rpt = NP // NS
  wb = rpt // CHUNK
  NG = J // GK

  def body(z_hbm, ep_hbm, out_hbm,
           ibufA, ibufB, rows0, rows1, acc, semiA, semiB,
           sem0, sem1, ssem0, ssem1):
    c = lax.axis_index("c")
    s = lax.axis_index("s")
    w = c * NS + s
    base = s * rpt
    pltpu.async_copy(ep_hbm.at[w].at[0], ibufA, semiA)

    def zr(r, carry):
      for k in range(D // 16):
        rows0[r, pl.ds(k * 16, 16)] = jnp.zeros((16,), jnp.float32)
      return carry

    lax.fori_loop(0, CHUNK, zr, 0)
    for k in range(wb):
      pltpu.sync_copy(rows0, acc.at[pl.ds(base + k * CHUNK, CHUNK)])
    plsc.subcore_barrier()

    rows = (rows0, rows1)
    gsem = (sem0, sem1)
    ssem = (ssem0, ssem1)

    pltpu.make_async_copy(ep_hbm.at[w].at[0], ibufA, semiA).wait()
    pltpu.async_copy(z_hbm.at[ibufA.at[0].at[0]], rows0, sem0)

    def half(cur, nxt, semi_nxt, nxt_g, guard, refill, first_wait_guard):
      for k in range(GK):
        p = k % 2
        rb, gs = rows[p], gsem[p]
        nrb, ngs = rows[1 - p], gsem[1 - p]
        pltpu.make_async_copy(z_hbm.at[cur.at[k].at[0]], rb, gs).wait()
        pltpu.make_async_copy(rb, acc.at[cur.at[k].at[1]], ssem[p]).start(add=True)
        def _retire_prev():
          pltpu.make_async_copy(nrb, acc.at[cur.at[k].at[1]], ssem[1 - p]).wait()
        if k == 0 and first_wait_guard is not None:
          pl.when(first_wait_guard)(_retire_prev)
        else:
          _retire_prev()
        if k == 0:
          tbuf, tsem, tg, tguard = refill
          def _refill():
            pltpu.async_copy(ep_hbm.at[w].at[tg], tbuf, tsem)
          if tguard is None:
            _refill()
          else:
            pl.when(tguard)(_refill)
        if k < GK - 1:
          pltpu.async_copy(z_hbm.at[cur.at[k + 1].at[0]], nrb, ngs)
        else:
          def _start_next():
            pltpu.make_async_copy(ep_hbm.at[w].at[nxt_g], nxt, semi_nxt).wait()
            pltpu.async_copy(z_hbm.at[nxt.at[0].at[0]], nrb, ngs)
          if guard is None:
            _start_next()
          else:
            pl.when(guard)(_start_next)

    def step(i, carry):
      more = i < NG // 2 - 1
      half(ibufA, ibufB, semiB, 2 * i + 1, None,
           (ibufB, semiB, 2 * i + 1, None), i > 0)
      half(ibufB, ibufA, semiA, 2 * i + 2, more,
           (ibufA, semiA, 2 * i + 2, more), None)
      return carry

    lax.fori_loop(0, NG // 2, step, 0)
    pltpu.make_async_copy(rows1, acc.at[ibufB.at[GK - 1].at[1]], ssem1).wait()
    plsc.subcore_barrier()
    for k in range(wb):
      sl = pl.ds(base + k * CHUNK, CHUNK)
      pltpu.sync_copy(acc.at[sl], rows0)
      pltpu.sync_copy(rows0, out_hbm.at[c].at[sl])

  return pl.kernel(
      body,
      out_type=jax.ShapeDtypeStruct((NC, NP, D), jnp.float32),
      mesh=_mesh(),
      scratch_types=[
          pltpu.VMEM((GK, 2, CHUNK), jnp.int32),
          pltpu.VMEM((GK, 2, CHUNK), jnp.int32),
          pltpu.VMEM((CHUNK, D), jnp.float32),
          pltpu.VMEM((CHUNK, D), jnp.float32),
          pltpu.VMEM_SHARED((NP, D), jnp.float32),
          pltpu.SemaphoreType.DMA,
          pltpu.SemaphoreType.DMA,
          pltpu.SemaphoreType.DMA,
          pltpu.SemaphoreType.DMA,
          pltpu.SemaphoreType.DMA,
          pltpu.SemaphoreType.DMA,
      ],
  )


def _dot(a, b):
  return jnp.dot(a, b, preferred_element_type=jnp.float32,
                 precision=lax.Precision.HIGHEST)


def _tc1(x_pad, W1, deg_parts):
  NP, D = x_pad.shape
  H = W1.shape[1]

  def body(x_ref, w_ref, dp_ref, z_ref, dinv_ref):
    deg = jnp.sum(dp_ref[...], axis=0) + 1.0
    dinv = lax.rsqrt(deg)
    z_ref[...] = _dot(x_ref[...], w_ref[...]) * dinv[:, None]
    dinv_ref[...] = dinv

  return pl.pallas_call(
      body,
      grid=(NP // BR,),
      in_specs=[
          pl.BlockSpec((BR, D), lambda i: (i, 0)),
          pl.BlockSpec((D, H), lambda i: (0, 0)),
          pl.BlockSpec((NC, BR), lambda i: (0, i)),
      ],
      out_specs=[
          pl.BlockSpec((BR, H), lambda i: (i, 0)),
          pl.BlockSpec((BR,), lambda i: (i,)),
      ],
      out_shape=[
          jax.ShapeDtypeStruct((NP, H), jnp.float32),
          jax.ShapeDtypeStruct((NP,), jnp.float32),
      ],
  )(x_pad, W1, deg_parts)


def _tc2(parts1, z1, dinv, b1, W2, n_valid):
  _, NP, D = parts1.shape
  H = W2.shape[1]

  def body(p_ref, z1_ref, dinv_ref, b1_ref, w2_ref, h1_ref, z2_ref):
    i = pl.program_id(0)
    sacc = p_ref[0] + p_ref[1] + z1_ref[...]
    dinv_c = dinv_ref[...][:, None]
    h1 = jnp.maximum(sacc * dinv_c + b1_ref[...][None, :], 0.0)
    h1_ref[...] = h1
    row = i * BR + lax.broadcasted_iota(jnp.int32, (BR, 1), 0)
    z2 = _dot(h1, w2_ref[...]) * dinv_c
    z2_ref[...] = jnp.where(row < n_valid, z2, 0.0)

  return pl.pallas_call(
      body,
      grid=(NP // BR,),
      in_specs=[
          pl.BlockSpec((NC, BR, D), lambda i: (0, i, 0)),
          pl.BlockSpec((BR, D), lambda i: (i, 0)),
          pl.BlockSpec((BR,), lambda i: (i,)),
          pl.BlockSpec((D,), lambda i: (0,)),
          pl.BlockSpec((D, H), lambda i: (0, 0)),
      ],
      out_specs=[
          pl.BlockSpec((BR, H), lambda i: (i, 0)),
          pl.BlockSpec((BR, H), lambda i: (i, 0)),
      ],
      out_shape=[
          jax.ShapeDtypeStruct((NP, H), jnp.float32),
          jax.ShapeDtypeStruct((NP, H), jnp.float32),
      ],
  )(parts1, z1, dinv, b1, W2)


def _tc3(parts2, z2, dinv, b2, h1, alpha, Wc_p, Wf_p, bc_p, bf_p):
  _, NP, D = parts2.shape

  def body(p_ref, z2_ref, dinv_ref, b2_ref, h1_ref, a_ref,
           wc_ref, wf_ref, bc_ref, bf_ref, out_ref):
    dinv_c = dinv_ref[...][:, None]
    h2 = jnp.maximum((p_ref[0] + p_ref[1] + z2_ref[...]) * dinv_c
                     + b2_ref[...][None, :], 0.0)
    a = a_ref[...][:, None]
    h1 = h1_ref[...]
    h2a = a * h2 + (1.0 - a) * h1
    lc = _dot(h1, wc_ref[...]) + bc_ref[...][None, :]
    lf = _dot(h2a, wf_ref[...]) + bf_ref[...][None, :]
    out_ref[...] = 0.5 * lc + 0.5 * lf

  return pl.pallas_call(
      body,
      grid=(NP // BR,),
      in_specs=[
          pl.BlockSpec((NC, BR, D), lambda i: (0, i, 0)),
          pl.BlockSpec((BR, D), lambda i: (i, 0)),
          pl.BlockSpec((BR,), lambda i: (i,)),
          pl.BlockSpec((D,), lambda i: (0,)),
          pl.BlockSpec((BR, D), lambda i: (i, 0)),
          pl.BlockSpec((BR,), lambda i: (i,)),
          pl.BlockSpec((D, LW), lambda i: (0, 0)),
          pl.BlockSpec((D, LW), lambda i: (0, 0)),
          pl.BlockSpec((LW,), lambda i: (0,)),
          pl.BlockSpec((LW,), lambda i: (0,)),
      ],
      out_specs=pl.BlockSpec((BR, LW), lambda i: (i, 0)),
      out_shape=jax.ShapeDtypeStruct((NP, LW), jnp.float32),
  )(parts2, z2, dinv, b2, h1, alpha, Wc_p, Wf_p, bc_p, bf_p)


def kernel(x, edge_index, h_node, W1, b1, Wc, bc, W2, b2, Wf, bf):
  N, D = x.shape
  H = W1.shape[1]
  E = edge_index.shape[1]
  C = Wc.shape[1]

  NP = -(-N // (NS * CHUNK)) * (NS * CHUNK)
  if NP == N:
    NP += NS * CHUNK
  J = -(-E // (NW * CHUNK))
  J = -(-J // (2 * GK)) * (2 * GK)
  EP = NW * J * CHUNK
  npad = EP - E

  src = edge_index[0].astype(jnp.int32)
  dst = edge_index[1].astype(jnp.int32)
  pad_idx = (N + jnp.arange(npad, dtype=jnp.int32) % (NP - N))
  srcp = jnp.concatenate([src, pad_idx]).reshape(NW, J, CHUNK)
  dstp = jnp.concatenate([dst, pad_idx]).reshape(NW, J, CHUNK)
  ep = jnp.stack([srcp, dstp], axis=2).reshape(NW, J // GK, GK, 2, CHUNK)

  x_pad = jnp.pad(x, ((0, NP - N), (0, 0)))
  a_pad = jnp.pad(h_node, (0, NP - N))
  Wc_p = jnp.pad(Wc, ((0, 0), (0, LW - C)))
  Wf_p = jnp.pad(Wf, ((0, 0), (0, LW - C)))
  bc_p = jnp.pad(bc, (0, LW - C))
  bf_p = jnp.pad(bf, (0, LW - C))

  deg_parts = _deg_kernel(NP, J)(dstp)
  z1, dinv = _tc1(x_pad, W1, deg_parts)
  seg = _edge_kernel(NP, J, H)
  parts1 = seg(z1, ep)
  h1, z2 = _tc2(parts1, z1, dinv, b1, W2, N)
  parts2 = seg(z2, ep)
  logits_pad = _tc3(parts2, z2, dinv, b2, h1, a_pad, Wc_p, Wf_p, bc_p, bf_p)
  return logits_pad[:N, :C]


# speedup vs baseline: 1.0648x; 1.0424x over previous
import functools

import jax
import jax.numpy as jnp
from jax import lax
from jax.experimental import pallas as pl
from jax.experimental.pallas import tpu as pltpu
from jax.experimental.pallas import tpu_sc as plsc

NC = 2
NS = 16
NW = NC * NS
CHUNK = 128
GK = 4
BR = 1024
LW = 128


def _mesh():
  return plsc.VectorSubcoreMesh(core_axis_name="c", subcore_axis_name="s")


@functools.lru_cache(maxsize=None)
def _deg_kernel(NP, J):
  rpt = NP // NS

  def body(dstp_hbm, out_hbm, dst_idx, ones_b, lin, deg_sh):
    c = lax.axis_index("c")
    s = lax.axis_index("s")
    w = c * NS + s
    base = s * rpt
    pltpu.sync_copy(dstp_hbm.at[w], dst_idx)
    for k in range(CHUNK // 16):
      ones_b[pl.ds(k * 16, 16)] = jnp.ones((16,), jnp.float32)
    for k in range(rpt // 16):
      lin[pl.ds(k * 16, 16)] = jnp.zeros((16,), jnp.float32)
    pltpu.sync_copy(lin, deg_sh.at[pl.ds(base, rpt)])
    plsc.subcore_barrier()

    def step(j, carry):
      pltpu.sync_copy(ones_b, deg_sh.at[dst_idx.at[j]], add=True)
      return carry

    lax.fori_loop(0, J, step, 0)
    plsc.subcore_barrier()
    pltpu.sync_copy(deg_sh.at[pl.ds(base, rpt)], lin)
    pltpu.sync_copy(lin, out_hbm.at[c].at[pl.ds(base, rpt)])

  return pl.kernel(
      body,
      out_type=jax.ShapeDtypeStruct((NC, NP), jnp.float32),
      mesh=_mesh(),
      scratch_types=[
          pltpu.VMEM((J, CHUNK), jnp.int32),
          pltpu.VMEM((CHUNK,), jnp.float32),
          pltpu.VMEM((rpt,), jnp.float32),
          pltpu.VMEM_SHARED((NP,), jnp.float32),
      ],
  )


@functools.lru_cache(maxsize=None)
def _edge_kernel(NP, J, D):
  rpt = NP // NS
  wb = rpt // CHUNK
  NG = J // GK

  def body(z_hbm, ep_hbm, out_hbm,
           ibufA, ibufB, rows0, rows1, acc, semiA, semiB,
           sem0, sem1, ssem0, ssem1):
    c = lax.axis_index("c")
    s = lax.axis_index("s")
    w = c * NS + s
    base = s * rpt
    pltpu.async_copy(ep_hbm.at[w].at[0], ibufA, semiA)

    def zr(r, carry):
      for k in range(D // 16):
        rows0[r, pl.ds(k * 16, 16)] = jnp.zeros((16,), jnp.float32)
      return carry

    lax.fori_loop(0, CHUNK, zr, 0)
    for k in range(wb):
      pltpu.sync_copy(rows0, acc.at[pl.ds(base + k * CHUNK, CHUNK)])
    plsc.subcore_barrier()

    rows = (rows0, rows1)
    gsem = (sem0, sem1)
    ssem = (ssem0, ssem1)

    pltpu.make_async_copy(ep_hbm.at[w].at[0], ibufA, semiA).wait()
    pltpu.async_copy(z_hbm.at[ibufA.at[0].at[0]], rows0, sem0)

    def half(cur, nxt, semi_nxt, nxt_g, guard, refill, first_wait_guard):
      for k in range(GK):
        p = k % 2
        rb, gs = rows[p], gsem[p]
        nrb, ngs = rows[1 - p], gsem[1 - p]
        pltpu.make_async_copy(z_hbm.at[cur.at[k].at[0]], rb, gs).wait()
        pltpu.make_async_copy(rb, acc.at[cur.at[k].at[1]], ssem[p]).start(add=True)
        def _retire_prev():
          pltpu.make_async_copy(nrb, acc.at[cur.at[k].at[1]], ssem[1 - p]).wait()
        if k == 0 and first_wait_guard is not None:
          pl.when(first_wait_guard)(_retire_prev)
        else:
          _retire_prev()
        if k == 0:
          tbuf, tsem, tg, tguard = refill
          def _refill():
            pltpu.async_copy(ep_hbm.at[w].at[tg], tbuf, tsem)
          if tguard is None:
            _refill()
          else:
            pl.when(tguard)(_refill)
        if k < GK - 1:
          pltpu.async_copy(z_hbm.at[cur.at[k + 1].at[0]], nrb, ngs)
        else:
          def _start_next():
            pltpu.make_async_copy(ep_hbm.at[w].at[nxt_g], nxt, semi_nxt).wait()
            pltpu.async_copy(z_hbm.at[nxt.at[0].at[0]], nrb, ngs)
          if guard is None:
            _start_next()
          else:
            pl.when(guard)(_start_next)

    def step(i, carry):
      more = i < NG // 2 - 1
      half(ibufA, ibufB, semiB, 2 * i + 1, None,
           (ibufB, semiB, 2 * i + 1, None), i > 0)
      half(ibufB, ibufA, semiA, 2 * i + 2, more,
           (ibufA, semiA, 2 * i + 2, more), None)
      return carry

    lax.fori_loop(0, NG // 2, step, 0)
    pltpu.make_async_copy(rows1, acc.at[ibufB.at[GK - 1].at[1]], ssem1).wait()
    plsc.subcore_barrier()
    for k in range(wb):
      sl = pl.ds(base + k * CHUNK, CHUNK)
      pltpu.sync_copy(acc.at[sl], rows0)
      pltpu.sync_copy(rows0, out_hbm.at[c].at[sl])

  return pl.kernel(
      body,
      out_type=jax.ShapeDtypeStruct((NC, NP, D), jnp.float32),
      mesh=_mesh(),
      scratch_types=[
          pltpu.VMEM((GK, 2, CHUNK), jnp.int32),
          pltpu.VMEM((GK, 2, CHUNK), jnp.int32),
          pltpu.VMEM((CHUNK, D), jnp.float32),
          pltpu.VMEM((CHUNK, D), jnp.float32),
          pltpu.VMEM_SHARED((NP, D), jnp.float32),
          pltpu.SemaphoreType.DMA,
          pltpu.SemaphoreType.DMA,
          pltpu.SemaphoreType.DMA,
          pltpu.SemaphoreType.DMA,
          pltpu.SemaphoreType.DMA,
          pltpu.SemaphoreType.DMA,
      ],
  )


def _dot(a, b):
  return jnp.dot(a, b, preferred_element_type=jnp.float32,
                 precision=lax.Precision.HIGHEST)


def _tc1(x_pad, W1, deg_parts):
  NP, D = x_pad.shape
  H = W1.shape[1]

  def body(x_ref, w_ref, dp_ref, z_ref, dinv_ref):
    deg = jnp.sum(dp_ref[...], axis=0) + 1.0
    dinv = lax.rsqrt(deg)
    z_ref[...] = _dot(x_ref[...], w_ref[...]) * dinv[:, None]
    dinv_ref[...] = dinv

  return pl.pallas_call(
      body,
      grid=(NP // BR,),
      in_specs=[
          pl.BlockSpec((BR, D), lambda i: (i, 0)),
          pl.BlockSpec((D, H), lambda i: (0, 0)),
          pl.BlockSpec((NC, BR), lambda i: (0, i)),
      ],
      out_specs=[
          pl.BlockSpec((BR, H), lambda i: (i, 0)),
          pl.BlockSpec((BR,), lambda i: (i,)),
      ],
      out_shape=[
          jax.ShapeDtypeStruct((NP, H), jnp.float32),
          jax.ShapeDtypeStruct((NP,), jnp.float32),
      ],
  )(x_pad, W1, deg_parts)


def _tc2(parts1, z1, dinv, b1, W2, n_valid):
  _, NP, D = parts1.shape
  H = W2.shape[1]

  def body(p_ref, z1_ref, dinv_ref, b1_ref, w2_ref, h1_ref, z2_ref):
    i = pl.program_id(0)
    sacc = p_ref[0] + p_ref[1] + z1_ref[...]
    dinv_c = dinv_ref[...][:, None]
    h1 = jnp.maximum(sacc * dinv_c + b1_ref[...][None, :], 0.0)
    h1_ref[...] = h1
    row = i * BR + lax.broadcasted_iota(jnp.int32, (BR, 1), 0)
    z2 = _dot(h1, w2_ref[...]) * dinv_c
    z2_ref[...] = jnp.where(row < n_valid, z2, 0.0)

  return pl.pallas_call(
      body,
      grid=(NP // BR,),
      in_specs=[
          pl.BlockSpec((NC, BR, D), lambda i: (0, i, 0)),
          pl.BlockSpec((BR, D), lambda i: (i, 0)),
          pl.BlockSpec((BR,), lambda i: (i,)),
          pl.BlockSpec((D,), lambda i: (0,)),
          pl.BlockSpec((D, H), lambda i: (0, 0)),
      ],
      out_specs=[
          pl.BlockSpec((BR, H), lambda i: (i, 0)),
          pl.BlockSpec((BR, H), lambda i: (i, 0)),
      ],
      out_shape=[
          jax.ShapeDtypeStruct((NP, H), jnp.float32),
          jax.ShapeDtypeStruct((NP, H), jnp.float32),
      ],
  )(parts1, z1, dinv, b1, W2)


def _tc3(parts2, z2, dinv, b2, h1, alpha, Wc_p, Wf_p, bc_p, bf_p):
  _, NP, D = parts2.shape

  def body(p_ref, z2_ref, dinv_ref, b2_ref, h1_ref, a_ref,
           wc_ref, wf_ref, bc_ref, bf_ref, out_ref):
    dinv_c = dinv_ref[...][:, None]
    h2 = jnp.maximum((p_ref[0] + p_ref[1] + z2_ref[...]) * dinv_c
                     + b2_ref[...][None, :], 0.0)
    a = a_ref[...][:, None]
    h1 = h1_ref[...]
    h2a = a * h2 + (1.0 - a) * h1
    lc = _dot(h1, wc_ref[...]) + bc_ref[...][None, :]
    lf = _dot(h2a, wf_ref[...]) + bf_ref[...][None, :]
    out_ref[...] = 0.5 * lc + 0.5 * lf

  return pl.pallas_call(
      body,
      grid=(NP // BR,),
      in_specs=[
          pl.BlockSpec((NC, BR, D), lambda i: (0, i, 0)),
          pl.BlockSpec((BR, D), lambda i: (i, 0)),
          pl.BlockSpec((BR,), lambda i: (i,)),
          pl.BlockSpec((D,), lambda i: (0,)),
          pl.BlockSpec((BR, D), lambda i: (i, 0)),
          pl.BlockSpec((BR,), lambda i: (i,)),
          pl.BlockSpec((D, LW), lambda i: (0, 0)),
          pl.BlockSpec((D, LW), lambda i: (0, 0)),
          pl.BlockSpec((LW,), lambda i: (0,)),
          pl.BlockSpec((LW,), lambda i: (0,)),
      ],
      out_specs=pl.BlockSpec((BR, LW), lambda i: (i, 0)),
      out_shape=jax.ShapeDtypeStruct((NP, LW), jnp.float32),
  )(parts2, z2, dinv, b2, h1, alpha, Wc_p, Wf_p, bc_p, bf_p)


def kernel(x, edge_index, h_node, W1, b1, Wc, bc, W2, b2, Wf, bf):
  N, D = x.shape
  H = W1.shape[1]
  E = edge_index.shape[1]
  C = Wc.shape[1]

  NP = -(-N // (NS * CHUNK)) * (NS * CHUNK)
  if NP == N:
    NP += NS * CHUNK
  J = -(-E // (NW * CHUNK))
  J = -(-J // (2 * GK)) * (2 * GK)
  EP = NW * J * CHUNK
  npad = EP - E

  src = edge_index[0].astype(jnp.int32)
  dst = edge_index[1].astype(jnp.int32)
  pad_idx = (N + jnp.arange(npad, dtype=jnp.int32) % (NP - N))
  srcp = jnp.concatenate([src, pad_idx]).reshape(NW, J, CHUNK)
  dstp = jnp.concatenate([dst, pad_idx]).reshape(NW, J, CHUNK)
  ep = jnp.stack([srcp, dstp], axis=2).reshape(NW, J // GK, GK, 2, CHUNK)

  x_pad = jnp.pad(x, ((0, NP - N), (0, 0)))
  a_pad = jnp.pad(h_node, (0, NP - N))
  Wc_p = jnp.pad(Wc, ((0, 0), (0, LW - C)))
  Wf_p = jnp.pad(Wf, ((0, 0), (0, LW - C)))
  bc_p = jnp.pad(bc, (0, LW - C))
  bf_p = jnp.pad(bf, (0, LW - C))

  deg_parts = _deg_kernel(NP, J)(dstp)
  z1, dinv = _tc1(x_pad, W1, deg_parts)
  seg = _edge_kernel(NP, J, H)
  parts1 = seg(z1, ep)
  h1, z2 = _tc2(parts1, z1, dinv, b1, W2, N)
  parts2 = seg(z2, ep)
  logits_pad = _tc3(parts2, z2, dinv, b2, h1, a_pad, Wc_p, Wf_p, bc_p, bf_p)
  return logits_pad[:N, :C]


# speedup vs baseline: 1.0892x; 1.0229x over previous
import functools

import jax
import jax.numpy as jnp
from jax import lax
from jax.experimental import pallas as pl
from jax.experimental.pallas import tpu as pltpu
from jax.experimental.pallas import tpu_sc as plsc

NC = 2
NS = 16
NW = NC * NS
CHUNK = 128
GK = 4
BR = 2048
LW = 128


def _mesh():
  return plsc.VectorSubcoreMesh(core_axis_name="c", subcore_axis_name="s")


@functools.lru_cache(maxsize=None)
def _deg_kernel(NP, J):
  rpt = NP // NS

  def body(dstp_hbm, out_hbm, dst_idx, ones_b, lin, deg_sh):
    c = lax.axis_index("c")
    s = lax.axis_index("s")
    w = c * NS + s
    base = s * rpt
    pltpu.sync_copy(dstp_hbm.at[w], dst_idx)
    for k in range(CHUNK // 16):
      ones_b[pl.ds(k * 16, 16)] = jnp.ones((16,), jnp.float32)
    for k in range(rpt // 16):
      lin[pl.ds(k * 16, 16)] = jnp.zeros((16,), jnp.float32)
    pltpu.sync_copy(lin, deg_sh.at[pl.ds(base, rpt)])
    plsc.subcore_barrier()

    def step(j, carry):
      pltpu.sync_copy(ones_b, deg_sh.at[dst_idx.at[j]], add=True)
      return carry

    lax.fori_loop(0, J, step, 0)
    plsc.subcore_barrier()
    pltpu.sync_copy(deg_sh.at[pl.ds(base, rpt)], lin)
    pltpu.sync_copy(lin, out_hbm.at[c].at[pl.ds(base, rpt)])

  return pl.kernel(
      body,
      out_type=jax.ShapeDtypeStruct((NC, NP), jnp.float32),
      mesh=_mesh(),
      scratch_types=[
          pltpu.VMEM((J, CHUNK), jnp.int32),
          pltpu.VMEM((CHUNK,), jnp.float32),
          pltpu.VMEM((rpt,), jnp.float32),
          pltpu.VMEM_SHARED((NP,), jnp.float32),
      ],
  )


@functools.lru_cache(maxsize=None)
def _edge_kernel(NP, J, D):
  rpt = NP // NS
  wb = rpt // CHUNK
  NG = J // GK

  def body(z_hbm, ep_hbm, out_hbm,
           ibufA, ibufB, rows0, rows1, acc, semiA, semiB,
           sem0, sem1, ssem0, ssem1):
    c = lax.axis_index("c")
    s = lax.axis_index("s")
    w = c * NS + s
    base = s * rpt
    pltpu.async_copy(ep_hbm.at[w].at[0], ibufA, semiA)

    def zr(r, carry):
      for k in range(D // 16):
        rows0[r, pl.ds(k * 16, 16)] = jnp.zeros((16,), jnp.float32)
      return carry

    lax.fori_loop(0, CHUNK, zr, 0)
    for k in range(wb):
      pltpu.sync_copy(rows0, acc.at[pl.ds(base + k * CHUNK, CHUNK)])
    plsc.subcore_barrier()

    rows = (rows0, rows1)
    gsem = (sem0, sem1)
    ssem = (ssem0, ssem1)

    pltpu.make_async_copy(ep_hbm.at[w].at[0], ibufA, semiA).wait()
    pltpu.async_copy(z_hbm.at[ibufA.at[0].at[0]], rows0, sem0)

    def half(cur, nxt, semi_nxt, nxt_g, guard, refill, first_wait_guard):
      for k in range(GK):
        p = k % 2
        rb, gs = rows[p], gsem[p]
        nrb, ngs = rows[1 - p], gsem[1 - p]
        pltpu.make_async_copy(z_hbm.at[cur.at[k].at[0]], rb, gs).wait()
        pltpu.make_async_copy(rb, acc.at[cur.at[k].at[1]], ssem[p]).start(add=True)
        def _retire_prev():
          pltpu.make_async_copy(nrb, acc.at[cur.at[k].at[1]], ssem[1 - p]).wait()
        if k == 0 and first_wait_guard is not None:
          pl.when(first_wait_guard)(_retire_prev)
        else:
          _retire_prev()
        if k == 0:
          tbuf, tsem, tg, tguard = refill
          def _refill():
            pltpu.async_copy(ep_hbm.at[w].at[tg], tbuf, tsem)
          if tguard is None:
            _refill()
          else:
            pl.when(tguard)(_refill)
        if k < GK - 1:
          pltpu.async_copy(z_hbm.at[cur.at[k + 1].at[0]], nrb, ngs)
        else:
          def _start_next():
            pltpu.make_async_copy(ep_hbm.at[w].at[nxt_g], nxt, semi_nxt).wait()
            pltpu.async_copy(z_hbm.at[nxt.at[0].at[0]], nrb, ngs)
          if guard is None:
            _start_next()
          else:
            pl.when(guard)(_start_next)

    def step(i, carry):
      more = i < NG // 2 - 1
      half(ibufA, ibufB, semiB, 2 * i + 1, None,
           (ibufB, semiB, 2 * i + 1, None), i > 0)
      half(ibufB, ibufA, semiA, 2 * i + 2, more,
           (ibufA, semiA, 2 * i + 2, more), None)
      return carry

    lax.fori_loop(0, NG // 2, step, 0)
    pltpu.make_async_copy(rows1, acc.at[ibufB.at[GK - 1].at[1]], ssem1).wait()
    plsc.subcore_barrier()
    for k in range(wb):
      sl = pl.ds(base + k * CHUNK, CHUNK)
      pltpu.sync_copy(acc.at[sl], rows0)
      pltpu.sync_copy(rows0, out_hbm.at[c].at[sl])

  return pl.kernel(
      body,
      out_type=jax.ShapeDtypeStruct((NC, NP, D), jnp.float32),
      mesh=_mesh(),
      scratch_types=[
          pltpu.VMEM((GK, 2, CHUNK), jnp.int32),
          pltpu.VMEM((GK, 2, CHUNK), jnp.int32),
          pltpu.VMEM((CHUNK, D), jnp.float32),
          pltpu.VMEM((CHUNK, D), jnp.float32),
          pltpu.VMEM_SHARED((NP, D), jnp.float32),
          pltpu.SemaphoreType.DMA,
          pltpu.SemaphoreType.DMA,
          pltpu.SemaphoreType.DMA,
          pltpu.SemaphoreType.DMA,
          pltpu.SemaphoreType.DMA,
          pltpu.SemaphoreType.DMA,
      ],
  )


def _dot(a, b):
  return jnp.dot(a, b, preferred_element_type=jnp.float32,
                 precision=lax.Precision.HIGHEST)


def _tc1(x_pad, W1, deg_parts):
  NP, D = x_pad.shape
  H = W1.shape[1]

  def body(x_ref, w_ref, dp_ref, z_ref, dinv_ref):
    deg = jnp.sum(dp_ref[...], axis=0) + 1.0
    dinv = lax.rsqrt(deg)
    z_ref[...] = _dot(x_ref[...], w_ref[...]) * dinv[:, None]
    dinv_ref[...] = dinv

  return pl.pallas_call(
      body,
      grid=(NP // BR,),
      in_specs=[
          pl.BlockSpec((BR, D), lambda i: (i, 0)),
          pl.BlockSpec((D, H), lambda i: (0, 0)),
          pl.BlockSpec((NC, BR), lambda i: (0, i)),
      ],
      out_specs=[
          pl.BlockSpec((BR, H), lambda i: (i, 0)),
          pl.BlockSpec((BR,), lambda i: (i,)),
      ],
      out_shape=[
          jax.ShapeDtypeStruct((NP, H), jnp.float32),
          jax.ShapeDtypeStruct((NP,), jnp.float32),
      ],
  )(x_pad, W1, deg_parts)


def _tc2(parts1, z1, dinv, b1, W2, n_valid):
  _, NP, D = parts1.shape
  H = W2.shape[1]

  def body(p_ref, z1_ref, dinv_ref, b1_ref, w2_ref, h1_ref, z2_ref):
    i = pl.program_id(0)
    sacc = p_ref[0] + p_ref[1] + z1_ref[...]
    dinv_c = dinv_ref[...][:, None]
    h1 = jnp.maximum(sacc * dinv_c + b1_ref[...][None, :], 0.0)
    h1_ref[...] = h1
    row = i * BR + lax.broadcasted_iota(jnp.int32, (BR, 1), 0)
    z2 = _dot(h1, w2_ref[...]) * dinv_c
    z2_ref[...] = jnp.where(row < n_valid, z2, 0.0)

  return pl.pallas_call(
      body,
      grid=(NP // BR,),
      in_specs=[
          pl.BlockSpec((NC, BR, D), lambda i: (0, i, 0)),
          pl.BlockSpec((BR, D), lambda i: (i, 0)),
          pl.BlockSpec((BR,), lambda i: (i,)),
          pl.BlockSpec((D,), lambda i: (0,)),
          pl.BlockSpec((D, H), lambda i: (0, 0)),
      ],
      out_specs=[
          pl.BlockSpec((BR, H), lambda i: (i, 0)),
          pl.BlockSpec((BR, H), lambda i: (i, 0)),
      ],
      out_shape=[
          jax.ShapeDtypeStruct((NP, H), jnp.float32),
          jax.ShapeDtypeStruct((NP, H), jnp.float32),
      ],
  )(parts1, z1, dinv, b1, W2)


def _tc3(parts2, z2, dinv, b2, h1, alpha, Wc_p, Wf_p, bc_p, bf_p):
  _, NP, D = parts2.shape

  def body(p_ref, z2_ref, dinv_ref, b2_ref, h1_ref, a_ref,
           wc_ref, wf_ref, bc_ref, bf_ref, out_ref):
    dinv_c = dinv_ref[...][:, None]
    h2 = jnp.maximum((p_ref[0] + p_ref[1] + z2_ref[...]) * dinv_c
                     + b2_ref[...][None, :], 0.0)
    a = a_ref[...][:, None]
    h1 = h1_ref[...]
    h2a = a * h2 + (1.0 - a) * h1
    lc = _dot(h1, wc_ref[...]) + bc_ref[...][None, :]
    lf = _dot(h2a, wf_ref[...]) + bf_ref[...][None, :]
    out_ref[...] = 0.5 * lc + 0.5 * lf

  return pl.pallas_call(
      body,
      grid=(NP // BR,),
      in_specs=[
          pl.BlockSpec((NC, BR, D), lambda i: (0, i, 0)),
          pl.BlockSpec((BR, D), lambda i: (i, 0)),
          pl.BlockSpec((BR,), lambda i: (i,)),
          pl.BlockSpec((D,), lambda i: (0,)),
          pl.BlockSpec((BR, D), lambda i: (i, 0)),
          pl.BlockSpec((BR,), lambda i: (i,)),
          pl.BlockSpec((D, LW), lambda i: (0, 0)),
          pl.BlockSpec((D, LW), lambda i: (0, 0)),
          pl.BlockSpec((LW,), lambda i: (0,)),
          pl.BlockSpec((LW,), lambda i: (0,)),
      ],
      out_specs=pl.BlockSpec((BR, LW), lambda i: (i, 0)),
      out_shape=jax.ShapeDtypeStruct((NP, LW), jnp.float32),
  )(parts2, z2, dinv, b2, h1, alpha, Wc_p, Wf_p, bc_p, bf_p)


def kernel(x, edge_index, h_node, W1, b1, Wc, bc, W2, b2, Wf, bf):
  N, D = x.shape
  H = W1.shape[1]
  E = edge_index.shape[1]
  C = Wc.shape[1]

  NP = -(-N // (NS * CHUNK)) * (NS * CHUNK)
  if NP == N:
    NP += NS * CHUNK
  J = -(-E // (NW * CHUNK))
  J = -(-J // (2 * GK)) * (2 * GK)
  EP = NW * J * CHUNK
  npad = EP - E

  src = edge_index[0].astype(jnp.int32)
  dst = edge_index[1].astype(jnp.int32)
  pad_idx = (N + jnp.arange(npad, dtype=jnp.int32) % (NP - N))
  srcp = jnp.concatenate([src, pad_idx]).reshape(NW, J, CHUNK)
  dstp = jnp.concatenate([dst, pad_idx]).reshape(NW, J, CHUNK)
  ep = jnp.stack([srcp, dstp], axis=2).reshape(NW, J // GK, GK, 2, CHUNK)

  x_pad = jnp.pad(x, ((0, NP - N), (0, 0)))
  a_pad = jnp.pad(h_node, (0, NP - N))
  Wc_p = jnp.pad(Wc, ((0, 0), (0, LW - C)))
  Wf_p = jnp.pad(Wf, ((0, 0), (0, LW - C)))
  bc_p = jnp.pad(bc, (0, LW - C))
  bf_p = jnp.pad(bf, (0, LW - C))

  deg_parts = _deg_kernel(NP, J)(dstp)
  z1, dinv = _tc1(x_pad, W1, deg_parts)
  seg = _edge_kernel(NP, J, H)
  parts1 = seg(z1, ep)
  h1, z2 = _tc2(parts1, z1, dinv, b1, W2, N)
  parts2 = seg(z2, ep)
  logits_pad = _tc3(parts2, z2, dinv, b2, h1, a_pad, Wc_p, Wf_p, bc_p, bf_p)
  return logits_pad[:N, :C]
